# baseline (speedup 1.0000x reference)
import jax
import jax.numpy as jnp
from jax import lax
from jax.experimental import pallas as pl
from jax.experimental.pallas import tpu as pltpu
from jax.experimental.pallas import tpu_sc as plsc

_B = 128
_R = 768
_C = 256
_BB = 4


def _gather_body(ts_hbm, ta_hbm, tb_hbm, a_out, c_out, idx_v, val_v, sem):
    wid = lax.axis_index("s") * 2 + lax.axis_index("c")

    @pl.when(wid == 0)
    def _():
        pltpu.sync_copy(ts_hbm, idx_v)
        pltpu.async_copy(ta_hbm.at[idx_v], val_v, sem).wait()
        pltpu.sync_copy(val_v, a_out)

    @pl.when(wid == 1)
    def _():
        pltpu.sync_copy(ts_hbm, idx_v)
        pltpu.async_copy(tb_hbm.at[idx_v], val_v, sem).wait()
        pltpu.sync_copy(val_v, c_out)


def _fma_body(a_ref, c_ref, o_ref, n_ref, out_ref):
    i = pl.program_id(0)
    for j in range(_BB):
        a = a_ref[i * _BB + j]
        c = c_ref[i * _BB + j]
        out_ref[j] = a * o_ref[j] + c * n_ref[j]


def kernel(original_samples, noise, timesteps, sqrt_alphas_cumprod,
           sqrt_one_minus_alphas_cumprod):
    orig = original_samples.reshape(_B, _R, _C)
    nz = noise.reshape(_B, _R, _C)
    ts = timesteps.astype(jnp.int32)

    mesh = plsc.VectorSubcoreMesh(core_axis_name="c", subcore_axis_name="s")
    a_coef, c_coef = pl.kernel(
        _gather_body,
        mesh=mesh,
        out_type=[
            jax.ShapeDtypeStruct((_B,), jnp.float32),
            jax.ShapeDtypeStruct((_B,), jnp.float32),
        ],
        scratch_types=[
            pltpu.VMEM((_B,), jnp.int32),
            pltpu.VMEM((_B,), jnp.float32),
            pltpu.SemaphoreType.DMA,
        ],
    )(ts, sqrt_alphas_cumprod, sqrt_one_minus_alphas_cumprod)

    grid_spec = pltpu.PrefetchScalarGridSpec(
        num_scalar_prefetch=2,
        grid=(_B // _BB,),
        in_specs=[
            pl.BlockSpec((_BB, _R, _C), lambda i, *_: (i, 0, 0)),
            pl.BlockSpec((_BB, _R, _C), lambda i, *_: (i, 0, 0)),
        ],
        out_specs=pl.BlockSpec((_BB, _R, _C), lambda i, *_: (i, 0, 0)),
    )

    out = pl.pallas_call(
        _fma_body,
        grid_spec=grid_spec,
        out_shape=jax.ShapeDtypeStruct((_B, _R, _C), jnp.float32),
        compiler_params=pltpu.CompilerParams(
            dimension_semantics=("arbitrary",)),
    )(a_coef, c_coef, orig, nz)

    return out.reshape(original_samples.shape)


# speedup vs baseline: 1.0091x; 1.0091x over previous
import jax
import jax.numpy as jnp
from jax import lax
from jax.experimental import pallas as pl
from jax.experimental.pallas import tpu as pltpu
from jax.experimental.pallas import tpu_sc as plsc

_B = 128
_R = 768
_C = 256
_BB = 4
_LO = 64
_HI = _B - _LO


def _gather_body(ts_hbm, ta_hbm, tb_hbm, a_out, c_out, idx_v, val_v, sem):
    wid = lax.axis_index("s") * 2 + lax.axis_index("c")

    @pl.when(wid == 0)
    def _():
        pltpu.sync_copy(ts_hbm, idx_v)
        pltpu.async_copy(ta_hbm.at[idx_v], val_v, sem).wait()
        pltpu.sync_copy(val_v, a_out)

    @pl.when(wid == 1)
    def _():
        pltpu.sync_copy(ts_hbm, idx_v)
        pltpu.async_copy(tb_hbm.at[idx_v], val_v, sem).wait()
        pltpu.sync_copy(val_v, c_out)


def _fma_lo_body(ts_ref, ta_ref, tb_ref, o_ref, n_ref, out_ref):
    i = pl.program_id(0)
    for j in range(_BB):
        t = ts_ref[i * _BB + j]
        a = ta_ref[t]
        c = tb_ref[t]
        out_ref[j] = a * o_ref[j] + c * n_ref[j]


def _fma_hi_body(a_ref, c_ref, prev_ref, o_ref, n_ref, out_ref):
    i = pl.program_id(0)
    for j in range(_BB):
        a = a_ref[i * _BB + j]
        c = c_ref[i * _BB + j]
        out_ref[j] = a * o_ref[j] + c * n_ref[j]


def kernel(original_samples, noise, timesteps, sqrt_alphas_cumprod,
           sqrt_one_minus_alphas_cumprod):
    orig = original_samples.reshape(_B, _R, _C)
    nz = noise.reshape(_B, _R, _C)
    ts = timesteps.astype(jnp.int32)

    mesh = plsc.VectorSubcoreMesh(core_axis_name="c", subcore_axis_name="s")
    a_hi, c_hi = pl.kernel(
        _gather_body,
        mesh=mesh,
        out_type=[
            jax.ShapeDtypeStruct((_HI,), jnp.float32),
            jax.ShapeDtypeStruct((_HI,), jnp.float32),
        ],
        scratch_types=[
            pltpu.VMEM((_HI,), jnp.int32),
            pltpu.VMEM((_HI,), jnp.float32),
            pltpu.SemaphoreType.DMA,
        ],
    )(ts[_LO:], sqrt_alphas_cumprod, sqrt_one_minus_alphas_cumprod)

    lo_spec = pltpu.PrefetchScalarGridSpec(
        num_scalar_prefetch=3,
        grid=(_LO // _BB,),
        in_specs=[
            pl.BlockSpec((_BB, _R, _C), lambda i, *_: (i, 0, 0)),
            pl.BlockSpec((_BB, _R, _C), lambda i, *_: (i, 0, 0)),
        ],
        out_specs=pl.BlockSpec((_BB, _R, _C), lambda i, *_: (i, 0, 0)),
    )
    out_lo = pl.pallas_call(
        _fma_lo_body,
        grid_spec=lo_spec,
        out_shape=jax.ShapeDtypeStruct((_B, _R, _C), jnp.float32),
        compiler_params=pltpu.CompilerParams(
            dimension_semantics=("arbitrary",)),
    )(ts[:_LO], sqrt_alphas_cumprod, sqrt_one_minus_alphas_cumprod,
      orig, nz)

    hi_spec = pltpu.PrefetchScalarGridSpec(
        num_scalar_prefetch=2,
        grid=(_HI // _BB,),
        in_specs=[
            pl.BlockSpec(memory_space=pl.ANY),
            pl.BlockSpec((_BB, _R, _C), lambda i, *_: (i + _LO // _BB, 0, 0)),
            pl.BlockSpec((_BB, _R, _C), lambda i, *_: (i + _LO // _BB, 0, 0)),
        ],
        out_specs=pl.BlockSpec((_BB, _R, _C), lambda i, *_: (i + _LO // _BB, 0, 0)),
    )
    out = pl.pallas_call(
        _fma_hi_body,
        grid_spec=hi_spec,
        out_shape=jax.ShapeDtypeStruct((_B, _R, _C), jnp.float32),
        input_output_aliases={2: 0},
        compiler_params=pltpu.CompilerParams(
            dimension_semantics=("arbitrary",)),
    )(a_hi, c_hi, out_lo, orig, nz)

    return out.reshape(original_samples.shape)


# speedup vs baseline: 1.0125x; 1.0034x over previous
import jax
import jax.numpy as jnp
from jax import lax
from jax.experimental import pallas as pl
from jax.experimental.pallas import tpu as pltpu
from jax.experimental.pallas import tpu_sc as plsc

_B = 128
_R = 768
_C = 256
_BB = 4
_LO = 64
_HI = _B - _LO


def _gather_body(ts_hbm, ta_hbm, tb_hbm, a_out, c_out, idx_v, val_v, sem):
    wid = lax.axis_index("s") * 2 + lax.axis_index("c")

    @pl.when(wid == 0)
    def _():
        pltpu.sync_copy(ts_hbm, idx_v)
        pltpu.async_copy(ta_hbm.at[idx_v], val_v, sem).wait()
        pltpu.sync_copy(val_v, a_out)

    @pl.when(wid == 1)
    def _():
        pltpu.sync_copy(ts_hbm, idx_v)
        pltpu.async_copy(tb_hbm.at[idx_v], val_v, sem).wait()
        pltpu.sync_copy(val_v, c_out)


def _fma_lo_body(ts_ref, ta_ref, tb_ref, o_ref, n_ref, out_ref):
    i = pl.program_id(0)
    for j in range(_BB):
        t = ts_ref[i * _BB + j]
        a = ta_ref[t]
        c = tb_ref[t]
        out_ref[j] = a * o_ref[j] + c * n_ref[j]


def _fma_hi_body(a_ref, c_ref, prev_ref, o_ref, n_ref, out_ref):
    i = pl.program_id(0)
    for j in range(_BB):
        a = a_ref[_LO + i * _BB + j]
        c = c_ref[_LO + i * _BB + j]
        out_ref[j] = a * o_ref[j] + c * n_ref[j]


def kernel(original_samples, noise, timesteps, sqrt_alphas_cumprod,
           sqrt_one_minus_alphas_cumprod):
    orig = original_samples.reshape(_B, _R, _C)
    nz = noise.reshape(_B, _R, _C)
    ts = timesteps.astype(jnp.int32)

    mesh = plsc.VectorSubcoreMesh(core_axis_name="c", subcore_axis_name="s")
    a_hi, c_hi = pl.kernel(
        _gather_body,
        mesh=mesh,
        out_type=[
            jax.ShapeDtypeStruct((_B,), jnp.float32),
            jax.ShapeDtypeStruct((_B,), jnp.float32),
        ],
        scratch_types=[
            pltpu.VMEM((_B,), jnp.int32),
            pltpu.VMEM((_B,), jnp.float32),
            pltpu.SemaphoreType.DMA,
        ],
    )(ts, sqrt_alphas_cumprod, sqrt_one_minus_alphas_cumprod)

    lo_spec = pltpu.PrefetchScalarGridSpec(
        num_scalar_prefetch=3,
        grid=(_LO // _BB,),
        in_specs=[
            pl.BlockSpec((_BB, _R, _C), lambda i, *_: (i, 0, 0)),
            pl.BlockSpec((_BB, _R, _C), lambda i, *_: (i, 0, 0)),
        ],
        out_specs=pl.BlockSpec((_BB, _R, _C), lambda i, *_: (i, 0, 0)),
    )
    out_lo = pl.pallas_call(
        _fma_lo_body,
        grid_spec=lo_spec,
        out_shape=jax.ShapeDtypeStruct((_B, _R, _C), jnp.float32),
        compiler_params=pltpu.CompilerParams(
            dimension_semantics=("arbitrary",)),
    )(ts, sqrt_alphas_cumprod, sqrt_one_minus_alphas_cumprod,
      orig, nz)

    hi_spec = pltpu.PrefetchScalarGridSpec(
        num_scalar_prefetch=2,
        grid=(_HI // _BB,),
        in_specs=[
            pl.BlockSpec(memory_space=pl.ANY),
            pl.BlockSpec((_BB, _R, _C), lambda i, *_: (i + _LO // _BB, 0, 0)),
            pl.BlockSpec((_BB, _R, _C), lambda i, *_: (i + _LO // _BB, 0, 0)),
        ],
        out_specs=pl.BlockSpec((_BB, _R, _C), lambda i, *_: (i + _LO // _BB, 0, 0)),
    )
    out = pl.pallas_call(
        _fma_hi_body,
        grid_spec=hi_spec,
        out_shape=jax.ShapeDtypeStruct((_B, _R, _C), jnp.float32),
        input_output_aliases={2: 0},
        compiler_params=pltpu.CompilerParams(
            dimension_semantics=("arbitrary",)),
    )(a_hi, c_hi, out_lo, orig, nz)

    return out.reshape(original_samples.shape)
